# baseline (speedup 1.0000x reference)
import functools

import jax
import jax.numpy as jnp
from jax import lax
from jax.experimental import pallas as pl
from jax.experimental.pallas import tpu as pltpu
from jax.experimental.pallas import tpu_sc as plsc

_MAX_DIST = 2048
_H = 16
_L = 16
_NC, _NS = 2, 16
_NW = _NC * _NS

_B, _T, _M = 4, 2048, 512
_ROWS = _B * _T * _M
_ROWS_PER_W = _ROWS // _NW
_T_PER_W = _ROWS_PER_W // _M
_V = 2 * _MAX_DIST - 1

_CR = _M
_NCH = _T_PER_W


def _body(q_hbm, k_hbm, table_hbm, out_hbm,
          q_loc, k_loc, tab, idxb, rows, ssem):
  wid = lax.axis_index("c") * _NS + lax.axis_index("s")
  b = wid // (_T // _T_PER_W)
  t0 = (wid % (_T // _T_PER_W)) * _T_PER_W
  row0 = wid * _ROWS_PER_W

  pltpu.sync_copy(q_hbm.at[pl.ds(t0, _T_PER_W)], q_loc.at[pl.ds(0, _T_PER_W)])
  pltpu.sync_copy(k_hbm.at[pl.ds(b * _M, _M)], k_loc)
  pltpu.sync_copy(table_hbm, tab)

  io16 = lax.iota(jnp.int32, _L) * _H

  def wait_scatter(d):
    pltpu.make_async_copy(rows.at[d], out_hbm.at[pl.ds(row0, _CR * _H)],
                          ssem).wait()

  def chunk(g, _):
    d = lax.rem(g, 2)
    qv = jnp.full((_L,), q_loc[pl.ds(g, _L)][0], dtype=jnp.int32)
    @pl.when(g >= 2)
    def _w():
      wait_scatter(d)
    iov = lax.iota(jnp.int32, _L)
    for mv in range(_M // _L):
      kv = k_loc[pl.ds(mv * _L, _L)]
      dd = qv - kv
      dd = jnp.minimum(jnp.maximum(dd, -(_MAX_DIST - 1)), _MAX_DIST - 1)
      idxb[pl.ds(mv * _L, _L)] = (dd + (_MAX_DIST - 1)) * _H
    for r0 in range(0, _CR, _L):
      svs = [plsc.load_gather(idxb, [jnp.full((_L,), r0 + i, jnp.int32)])
             for i in range(_L)]
      rws = [plsc.load_gather(tab, [svs[i] + iov]) for i in range(_L)]
      for i in range(_L):
        rows[d, pl.ds((r0 + i) * _H, _H)] = rws[i]
    pltpu.async_copy(rows.at[d], out_hbm.at[pl.ds(row0 + g * _CR * _H,
                                                  _CR * _H)], ssem)
    return _

  lax.fori_loop(0, _NCH, chunk, 0)
  wait_scatter(0)
  wait_scatter(1)


@jax.jit
def _run(q_pos, k_pos, bias_table):
  mesh = plsc.VectorSubcoreMesh(core_axis_name="c", subcore_axis_name="s")
  out = pl.kernel(
      _body,
      out_type=jax.ShapeDtypeStruct((_ROWS * _H,), jnp.float32),
      mesh=mesh,
      compiler_params=pltpu.CompilerParams(needs_layout_passes=False),
      scratch_types=[
          pltpu.VMEM((_T_PER_W + _L,), jnp.int32),
          pltpu.VMEM((_M,), jnp.int32),
          pltpu.VMEM((_V * _H,), jnp.float32),
          pltpu.VMEM((_M,), jnp.int32),
          pltpu.VMEM((2, _CR * _H), jnp.float32),
          pltpu.SemaphoreType.DMA,
      ],
  )(q_pos, k_pos.reshape(_B * _M), bias_table.reshape(_V * _H))
  return out.reshape(_B, _T, _M, _H)


def kernel(q_pos, k_pos, bias_table):
  return _run(q_pos.astype(jnp.int32), k_pos.astype(jnp.int32), bias_table)


# speedup vs baseline: 1.0465x; 1.0465x over previous
import functools

import jax
import jax.numpy as jnp
from jax import lax
from jax.experimental import layout as jax_layout
from jax.experimental import pallas as pl
from jax.experimental.pallas import tpu as pltpu
from jax.experimental.pallas import tpu_sc as plsc

_MAX_DIST = 2048
_H = 16
_L = 16
_NC, _NS = 2, 16
_NW = _NC * _NS

_B, _T, _M = 4, 2048, 512
_ROWS = _B * _T * _M
_ROWS_PER_W = _ROWS // _NW
_T_PER_W = _ROWS_PER_W // _M
_V = 2 * _MAX_DIST - 1

_CR = _M
_NCH = _T_PER_W


def _body(q_hbm, k_hbm, table_hbm, out_hbm,
          q_loc, k_loc, tab, idxb, rows, ssem):
  wid = lax.axis_index("c") * _NS + lax.axis_index("s")
  b = wid // (_T // _T_PER_W)
  t0 = (wid % (_T // _T_PER_W)) * _T_PER_W
  row0 = wid * _ROWS_PER_W

  pltpu.sync_copy(q_hbm.at[pl.ds(t0, _T_PER_W)], q_loc.at[pl.ds(0, _T_PER_W)])
  pltpu.sync_copy(k_hbm.at[pl.ds(b * _M, _M)], k_loc)
  pltpu.sync_copy(table_hbm, tab)

  io16 = lax.iota(jnp.int32, _L) * _H

  def wait_scatter(d):
    pltpu.make_async_copy(rows.at[d], out_hbm.at[pl.ds(row0, _CR * _H)],
                          ssem).wait()

  def chunk(g, _):
    d = lax.rem(g, 2)
    qv = jnp.full((_L,), q_loc[pl.ds(g, _L)][0], dtype=jnp.int32)
    @pl.when(g >= 2)
    def _w():
      wait_scatter(d)
    iov = lax.iota(jnp.int32, _L)
    for mv in range(_M // _L):
      kv = k_loc[pl.ds(mv * _L, _L)]
      dd = qv - kv
      dd = jnp.minimum(jnp.maximum(dd, -(_MAX_DIST - 1)), _MAX_DIST - 1)
      idxb[pl.ds(mv * _L, _L)] = (dd + (_MAX_DIST - 1)) * _H
    for r0 in range(0, _CR, _L):
      svs = [plsc.load_gather(idxb, [jnp.full((_L,), r0 + i, jnp.int32)])
             for i in range(_L)]
      rws = [plsc.load_gather(tab, [svs[i] + iov]) for i in range(_L)]
      for i in range(_L):
        rows[d, pl.ds((r0 + i) * _H, _H)] = rws[i]
    pltpu.async_copy(rows.at[d], out_hbm.at[pl.ds(row0 + g * _CR * _H,
                                                  _CR * _H)], ssem)
    return _

  lax.fori_loop(0, _NCH, chunk, 0)
  wait_scatter(0)
  wait_scatter(1)


def _run(q_pos, k_pos, bias_table):
  mesh = plsc.VectorSubcoreMesh(core_axis_name="c", subcore_axis_name="s")
  out = pl.kernel(
      _body,
      out_type=jax.ShapeDtypeStruct((_ROWS * _H,), jnp.float32),
      mesh=mesh,
      compiler_params=pltpu.CompilerParams(use_tc_tiling_on_sc=False,
                                           needs_layout_passes=False),
      scratch_types=[
          pltpu.VMEM((_T_PER_W + _L,), jnp.int32),
          pltpu.VMEM((_M,), jnp.int32),
          pltpu.VMEM((_V * _H,), jnp.float32),
          pltpu.VMEM((_M,), jnp.int32),
          pltpu.VMEM((2, _CR * _H), jnp.float32),
          pltpu.SemaphoreType.DMA,
      ],
  )(q_pos, k_pos.reshape(_B * _M), bias_table.reshape(_V * _H))
  return out.reshape(_B, _T, _M, _H)


_jitted = None


def kernel(q_pos, k_pos, bias_table):
  global _jitted
  if _jitted is None:
    fmt = jax_layout.Format(
        jax_layout.Layout(major_to_minor=(3, 2, 1, 0), tiling=()),
        jax.sharding.SingleDeviceSharding(jax.devices()[0]))
    _jitted = jax.jit(_run, out_shardings=fmt)
  return _jitted(q_pos.astype(jnp.int32), k_pos.astype(jnp.int32), bias_table)


# speedup vs baseline: 1.0473x; 1.0008x over previous
import functools

import jax
import jax.numpy as jnp
from jax import lax
from jax.experimental import pallas as pl
from jax.experimental.pallas import tpu as pltpu
from jax.experimental.pallas import tpu_sc as plsc

_MAX_DIST = 2048
_H = 16
_L = 16
_NC, _NS = 2, 16
_NW = _NC * _NS

_B, _T, _M = 4, 2048, 512
_ROWS = _B * _T * _M
_ROWS_PER_W = _ROWS // _NW
_T_PER_W = _ROWS_PER_W // _M
_V = 2 * _MAX_DIST - 1

_CR = _M
_NCH = _T_PER_W
_NBUF = 4


def _body(q_hbm, k_hbm, table_hbm, out_hbm,
          q_loc, k_loc, tab, idxb, rows, ssem):
  wid = lax.axis_index("c") * _NS + lax.axis_index("s")
  b = wid // (_T // _T_PER_W)
  t0 = (wid % (_T // _T_PER_W)) * _T_PER_W
  row0 = wid * _ROWS_PER_W

  pltpu.sync_copy(q_hbm.at[pl.ds(t0, _T_PER_W)], q_loc.at[pl.ds(0, _T_PER_W)])
  pltpu.sync_copy(k_hbm.at[pl.ds(b * _M, _M)], k_loc)
  pltpu.sync_copy(table_hbm, tab)

  io16 = lax.iota(jnp.int32, _L) * _H

  def wait_scatter(d):
    pltpu.make_async_copy(rows.at[d], out_hbm.at[pl.ds(row0, _CR * _H)],
                          ssem).wait()

  def chunk(g, _):
    d = lax.rem(g, _NBUF)
    qv = jnp.full((_L,), q_loc[pl.ds(g, _L)][0], dtype=jnp.int32)
    @pl.when(g >= _NBUF)
    def _w():
      wait_scatter(d)
    iov = lax.iota(jnp.int32, _L)
    for mv in range(_M // _L):
      kv = k_loc[pl.ds(mv * _L, _L)]
      dd = qv - kv
      dd = jnp.minimum(jnp.maximum(dd, -(_MAX_DIST - 1)), _MAX_DIST - 1)
      idxb[pl.ds(mv * _L, _L)] = (dd + (_MAX_DIST - 1)) * _H
    for r0 in range(0, _CR, _L):
      svs = [plsc.load_gather(idxb, [jnp.full((_L,), r0 + i, jnp.int32)])
             for i in range(_L)]
      rws = [plsc.load_gather(tab, [svs[i] + iov]) for i in range(_L)]
      for i in range(_L):
        rows[d, pl.ds((r0 + i) * _H, _H)] = rws[i]
    pltpu.async_copy(rows.at[d], out_hbm.at[pl.ds(row0 + g * _CR * _H,
                                                  _CR * _H)], ssem)
    return _

  lax.fori_loop(0, _NCH, chunk, 0)
  for d in range(_NBUF):
    wait_scatter(d)


@jax.jit
def _run(q_pos, k_pos, bias_table):
  mesh = plsc.VectorSubcoreMesh(core_axis_name="c", subcore_axis_name="s")
  out = pl.kernel(
      _body,
      out_type=jax.ShapeDtypeStruct((_ROWS * _H,), jnp.float32),
      mesh=mesh,
      compiler_params=pltpu.CompilerParams(use_tc_tiling_on_sc=False,
                                           needs_layout_passes=False),
      scratch_types=[
          pltpu.VMEM((_T_PER_W + _L,), jnp.int32),
          pltpu.VMEM((_M,), jnp.int32),
          pltpu.VMEM((_V * _H,), jnp.float32),
          pltpu.VMEM((_M,), jnp.int32),
          pltpu.VMEM((_NBUF, _CR * _H), jnp.float32),
          pltpu.SemaphoreType.DMA,
      ],
  )(q_pos, k_pos.reshape(_B * _M), bias_table.reshape(_V * _H))
  return out.reshape(_B, _T, _M, _H)


def kernel(q_pos, k_pos, bias_table):
  return _run(q_pos.astype(jnp.int32), k_pos.astype(jnp.int32), bias_table)


# speedup vs baseline: 1.0888x; 1.0396x over previous
import functools

import jax
import jax.numpy as jnp
from jax import lax
from jax.experimental import pallas as pl
from jax.experimental.pallas import tpu as pltpu
from jax.experimental.pallas import tpu_sc as plsc

_MAX_DIST = 2048
_H = 16
_L = 16
_NC, _NS = 2, 16
_NW = _NC * _NS

_B, _T, _M = 4, 2048, 512
_ROWS = _B * _T * _M
_ROWS_PER_W = _ROWS // _NW
_T_PER_W = _ROWS_PER_W // _M
_V = 2 * _MAX_DIST - 1

_GSZ = 128
_TC = 4
_CR = _TC * _M
_NG = _CR // _GSZ
_NCH = _T_PER_W // _TC


def _body(q_hbm, k_hbm, table_hbm, out_hbm,
          q_loc, k_loc, stab, idx_buf, rows, gsem, ssem):
  wid = lax.axis_index("c") * _NS + lax.axis_index("s")
  b = wid // (_T // _T_PER_W)
  t0 = (wid % (_T // _T_PER_W)) * _T_PER_W
  row0 = wid * _ROWS_PER_W

  pltpu.sync_copy(q_hbm.at[pl.ds(t0, _T_PER_W)], q_loc.at[pl.ds(0, _T_PER_W)])
  pltpu.sync_copy(k_hbm.at[pl.ds(b * _M, _M)], k_loc)
  @pl.when(lax.axis_index("s") == 0)
  def _stage():
    pltpu.sync_copy(table_hbm, stab)
  plsc.subcore_barrier()

  def fire_gathers(d):
    for j in range(_NG):
      pltpu.async_copy(stab.at[idx_buf.at[d, j]],
                       rows.at[d, pl.ds(j * _GSZ, _GSZ)], gsem)

  def drain_gathers(d):
    pltpu.make_async_copy(out_hbm.at[pl.ds(row0, _CR)], rows.at[d], gsem).wait()

  def fire_scatter(d, g):
    pltpu.async_copy(rows.at[d], out_hbm.at[pl.ds(row0 + g * _CR, _CR)], ssem)

  def wait_scatter(d):
    pltpu.make_async_copy(rows.at[d], out_hbm.at[pl.ds(row0, _CR)], ssem).wait()

  def compute_idx(g, d):
    for tt in range(_TC):
      qv = jnp.full((_L,), q_loc[pl.ds(g * _TC + tt, _L)][0], dtype=jnp.int32)
      for mv in range(_M // _L):
        kv = k_loc[pl.ds(mv * _L, _L)]
        dd = qv - kv
        dd = jnp.minimum(jnp.maximum(dd, -(_MAX_DIST - 1)), _MAX_DIST - 1)
        p = tt * _M + mv * _L
        idx_buf[d, p // _GSZ, pl.ds(p % _GSZ, _L)] = dd + (_MAX_DIST - 1)

  def chunk(g, _):
    d = lax.rem(g, 2)
    @pl.when(g >= 2)
    def _w():
      wait_scatter(d)
    compute_idx(g, d)
    fire_gathers(d)
    @pl.when(g >= 1)
    def _s():
      drain_gathers(1 - d)
      fire_scatter(1 - d, g - 1)
    return _

  lax.fori_loop(0, _NCH, chunk, 0)
  dl = (_NCH - 1) % 2
  drain_gathers(dl)
  fire_scatter(dl, _NCH - 1)
  wait_scatter(0)
  wait_scatter(1)


@jax.jit
def _run(q_pos, k_pos, bias_table):
  mesh = plsc.VectorSubcoreMesh(core_axis_name="c", subcore_axis_name="s")
  out = pl.kernel(
      _body,
      out_type=jax.ShapeDtypeStruct((_ROWS, _H), jnp.float32),
      mesh=mesh,
      compiler_params=pltpu.CompilerParams(use_tc_tiling_on_sc=False),
      scratch_types=[
          pltpu.VMEM((_T_PER_W + _L,), jnp.int32),
          pltpu.VMEM((_M,), jnp.int32),
          pltpu.VMEM_SHARED((_V, _H), jnp.float32),
          pltpu.VMEM((2, _NG, _GSZ), jnp.int32),
          pltpu.VMEM((2, _CR, _H), jnp.float32),
          pltpu.SemaphoreType.DMA,
          pltpu.SemaphoreType.DMA,
      ],
  )(q_pos, k_pos.reshape(_B * _M), bias_table)
  return out.reshape(_B, _T, _M, _H)


def kernel(q_pos, k_pos, bias_table):
  return _run(q_pos.astype(jnp.int32), k_pos.astype(jnp.int32), bias_table)
